# Initial kernel scaffold; baseline (speedup 1.0000x reference)
import jax
import jax.numpy as jnp
from jax.experimental import pallas as pl


def kernel(x, edge_index, num_nodes, W, b):
    raise NotImplementedError("write your pallas kernel here")


# speedup vs baseline: 18.2912x; 18.2912x over previous
import functools

import jax
import jax.numpy as jnp
from jax import lax
from jax.experimental import pallas as pl
from jax.experimental.pallas import tpu as pltpu
from jax.experimental.pallas import tpu_sc as plsc

N = 10000
D = 128
NP = 10240
NC = 2
NS = 16
NW = NC * NS
CH = 128
NCHUNK = 79
EPT = NCHUNK * CH
DUMMY = 10200
RPT = NP // NS

_mesh = plsc.VectorSubcoreMesh(core_axis_name="c", subcore_axis_name="s")


@functools.partial(
    pl.kernel,
    mesh=_mesh,
    out_type=jax.ShapeDtypeStruct((NC, NP, 16), jnp.float32),
    scratch_types=[
        pltpu.VMEM((NCHUNK, CH), jnp.int32),
        pltpu.VMEM((CH, 16), jnp.float32),
        pltpu.VMEM((RPT, 16), jnp.float32),
        pltpu.VMEM_SHARED((NP, 16), jnp.float32),
    ],
)
def _hist(dst_hbm, deg_out, dst_v, ones_v, zero_v, deg_sp):
    c = lax.axis_index("c")
    s = lax.axis_index("s")
    wid = c * NS + s

    def fill(r, carry):
        ones_v[r, :] = jnp.ones((16,), jnp.float32)
        return carry

    lax.fori_loop(0, CH, fill, 0)

    def zfill(r, carry):
        zero_v[r, :] = jnp.zeros((16,), jnp.float32)
        return carry

    lax.fori_loop(0, RPT, zfill, 0)
    pltpu.sync_copy(zero_v, deg_sp.at[pl.ds(0, RPT)])
    pltpu.sync_copy(dst_hbm.at[wid], dst_v)
    plsc.subcore_barrier()

    def add_chunk(j, carry):
        pltpu.sync_copy(ones_v, deg_sp.at[dst_v.at[j]], add=True)
        return carry

    lax.fori_loop(0, NCHUNK, add_chunk, 0)
    plsc.subcore_barrier()
    pltpu.sync_copy(deg_sp.at[pl.ds(0, RPT)], zero_v)
    pltpu.sync_copy(zero_v, deg_out.at[c, pl.ds(s * RPT, RPT)])


@functools.partial(
    pl.kernel,
    mesh=_mesh,
    out_type=jax.ShapeDtypeStruct((NC, NP, D), jnp.float32),
    scratch_types=[
        pltpu.VMEM((NCHUNK, CH), jnp.int32),
        pltpu.VMEM((NCHUNK, CH), jnp.int32),
        pltpu.VMEM((CH, D), jnp.float32),
        pltpu.SemaphoreType.DMA,
        pltpu.VMEM_SHARED((NP, D), jnp.float32),
    ],
)
def _gather_scatter(xs_hbm, src_hbm, dst_hbm, s_out,
                    src_v, dst_v, row_a, sem_a, s_sp):
    c = lax.axis_index("c")
    s = lax.axis_index("s")
    wid = c * NS + s

    def zfill(r, carry):
        for k in range(D // 16):
            row_a[r, pl.ds(k * 16, 16)] = jnp.zeros((16,), jnp.float32)
        return carry

    lax.fori_loop(0, CH, zfill, 0)
    for t in range(RPT // CH):
        pltpu.sync_copy(row_a, s_sp.at[pl.ds(t * CH, CH)])
    pltpu.sync_copy(src_hbm.at[wid], src_v)
    pltpu.sync_copy(dst_hbm.at[wid], dst_v)
    plsc.subcore_barrier()

    def body(j, carry):
        pltpu.async_copy(xs_hbm.at[src_v.at[j]], row_a, sem_a).wait()
        pltpu.sync_copy(row_a, s_sp.at[dst_v.at[j]], add=True)
        return carry

    lax.fori_loop(0, NCHUNK, body, 0)
    plsc.subcore_barrier()
    for t in range(RPT // CH):
        pltpu.sync_copy(s_sp.at[pl.ds(t * CH, CH)], row_a)
        pltpu.sync_copy(row_a, s_out.at[c, pl.ds(s * RPT + t * CH, CH)])


def _prescale_body(deg_ref, x_ref, xs_ref):
    d = deg_ref[0, :, 0:1] + deg_ref[1, :, 0:1] + 1.0
    xs_ref[...] = x_ref[...] * lax.rsqrt(d)


def _finish_body(xs_ref, s_ref, deg_ref, w_ref, b_ref, o_ref):
    d = deg_ref[0, :, 0:1] + deg_ref[1, :, 0:1] + 1.0
    agg = (xs_ref[...] + s_ref[0] + s_ref[1]) * lax.rsqrt(d)
    acc = lax.dot_general(agg, w_ref[...], (((1,), (1,)), ((), ())),
                          preferred_element_type=jnp.float32)
    o_ref[...] = jnp.maximum(acc + b_ref[...], 0.0)


_RB = 1000


def kernel(x, edge_index, num_nodes, W, b):
    del num_nodes
    ei = edge_index.astype(jnp.int32)
    src = ei[1]
    dst = ei[0]
    e = src.shape[0]
    pad = NW * EPT - e
    src_p = jnp.concatenate(
        [src, jnp.zeros((pad,), jnp.int32)]).reshape(NW, NCHUNK, CH)
    dst_p = jnp.concatenate(
        [dst, jnp.full((pad,), DUMMY, jnp.int32)]).reshape(NW, NCHUNK, CH)

    deg = _hist(dst_p)

    xs = pl.pallas_call(
        _prescale_body,
        grid=(N // _RB,),
        in_specs=[
            pl.BlockSpec((NC, _RB, 16), lambda i: (0, i, 0)),
            pl.BlockSpec((_RB, D), lambda i: (i, 0)),
        ],
        out_specs=pl.BlockSpec((_RB, D), lambda i: (i, 0)),
        out_shape=jax.ShapeDtypeStruct((N, D), jnp.float32),
    )(deg, x)

    s_part = _gather_scatter(xs, src_p, dst_p)

    out = pl.pallas_call(
        _finish_body,
        grid=(N // _RB,),
        in_specs=[
            pl.BlockSpec((_RB, D), lambda i: (i, 0)),
            pl.BlockSpec((NC, _RB, D), lambda i: (0, i, 0)),
            pl.BlockSpec((NC, _RB, 16), lambda i: (0, i, 0)),
            pl.BlockSpec((D, D), lambda i: (0, 0)),
            pl.BlockSpec((1, D), lambda i: (0, 0)),
        ],
        out_specs=pl.BlockSpec((_RB, D), lambda i: (i, 0)),
        out_shape=jax.ShapeDtypeStruct((N, D), jnp.float32),
    )(xs, s_part, deg, W, b.reshape(1, D))
    return out
